# baseline (speedup 1.0000x reference)
import jax
import jax.numpy as jnp
from jax import lax
from jax.experimental import pallas as pl
from jax.experimental.pallas import tpu as pltpu
from jax.experimental.pallas import tpu_sc as plsc

B = 16384
L = 50
D = 64
NE = 1000000
NC = 2
NS = 16
NW = NC * NS
RPW = B // NW
CB = 2
CIDX = CB * L
NCHUNK = RPW // CB

PERM = [32 * (f // 32) + (f % 32) // 2 + 16 * (f % 2) for f in range(D)]


def _gather_body(idx_hbm, table_hbm, out_hbm, idx_v, buf0, buf1, out_v,
                 sem0, sem1):
    wid = lax.axis_index("s") * NC + lax.axis_index("c")
    pltpu.sync_copy(idx_hbm.at[wid], idx_v)
    bufs = (buf0, buf1)
    sems = (sem0, sem1)

    for b in range(2):
        pltpu.async_copy(table_hbm.at[idx_v.at[b]], bufs[b], sems[b])

    inv = jnp.float32(1.0 / L)

    def reduce_chunk(c, src):
        for r in range(CB):
            j0 = r * L
            accs = [jnp.zeros((16,), jnp.float32) for _ in range(4)]
            for j in range(L):
                for g in range(2):
                    v = src[j0 + j, pl.ds(32 * g, 32)]
                    xlo, xhi = plsc.unpack(
                        v, format=plsc.PackFormat.INTERLEAVED)
                    accs[2 * g] = accs[2 * g] + xlo
                    accs[2 * g + 1] = accs[2 * g + 1] + xhi
            row = c * CB + r
            for d in range(4):
                out_v[row, pl.ds(d * 16, 16)] = accs[d] * inv

    @pl.loop(0, NCHUNK // 2)
    def _chunks(c0):
        for b in range(2):
            c = c0 * 2 + b
            pltpu.make_async_copy(
                table_hbm.at[idx_v.at[c]], bufs[b], sems[b]).wait()
            reduce_chunk(c, bufs[b])
            nxt = c + 2

            @pl.when(nxt < NCHUNK)
            def _():
                pltpu.async_copy(table_hbm.at[idx_v.at[nxt]], bufs[b], sems[b])

    pltpu.sync_copy(out_v, out_hbm.at[pl.ds(wid * RPW, RPW)])


def kernel(x, table):
    tb = table.astype(jnp.bfloat16)
    idx = x.astype(jnp.int32).reshape(NW, NCHUNK, CIDX)
    mesh = plsc.VectorSubcoreMesh(
        core_axis_name="c", subcore_axis_name="s",
        num_cores=NC, num_subcores=NS)
    gather = pl.kernel(
        _gather_body,
        out_type=jax.ShapeDtypeStruct((B, D), jnp.float32),
        mesh=mesh,
        scratch_types=[
            pltpu.VMEM((NCHUNK, CIDX), jnp.int32),
            pltpu.VMEM((CIDX, D), jnp.bfloat16),
            pltpu.VMEM((CIDX, D), jnp.bfloat16),
            pltpu.VMEM((RPW, D), jnp.float32),
            pltpu.SemaphoreType.DMA,
            pltpu.SemaphoreType.DMA,
        ],
        compiler_params=pltpu.CompilerParams(
            use_tc_tiling_on_sc=False, needs_layout_passes=False),
    )
    out_raw = gather(idx, tb)
    return out_raw[:, jnp.array(PERM)]
